# baseline (speedup 1.0000x reference)
import jax
import jax.numpy as jnp
from jax import lax
from jax.experimental import pallas as pl
from jax.experimental.pallas import tpu as pltpu
from jax.experimental.pallas import tpu_sc as plsc

NC = 2
NS = 16

BATCH = 4096
HIST = 200
VOCAB = 1000
D = 64
N_IDX = BATCH * HIST
N_PER_SC = N_IDX // NC

R_ROWS = 6400
N_ROUNDS = N_PER_SC // R_ROWS
T_ROWS = R_ROWS // NS
H_ROWS = T_ROWS // 2
S_ROWS = R_ROWS // 2
NBUF = 2


def _body(tokens_hbm, table_hbm, out_hbm, idx_v, local_v, shared,
          isem, gsem, csem, dwsem, wsem):
    c = lax.axis_index("c")
    s = lax.axis_index("s")

    def idxload(r):
        return pltpu.make_async_copy(
            tokens_hbm.at[c, r, s],
            idx_v.at[lax.rem(r, 4)],
            isem.at[lax.rem(r, 4)],
        )

    def gather(r):
        return pltpu.make_async_copy(
            table_hbm.at[idx_v.at[lax.rem(r, 4)]],
            local_v.at[lax.rem(r, 3)],
            gsem.at[lax.rem(r, 3)],
        )

    def copy(r):
        return pltpu.make_async_copy(
            local_v.at[lax.rem(r, 3), pl.ds(0, H_ROWS)],
            shared.at[lax.rem(r, NBUF), pl.ds(s * H_ROWS, H_ROWS)],
            csem,
        )

    def dwrite(r):
        return pltpu.make_async_copy(
            local_v.at[lax.rem(r, 3), pl.ds(H_ROWS, H_ROWS)],
            out_hbm.at[pl.ds((c * N_ROUNDS + r) * R_ROWS + S_ROWS
                             + s * H_ROWS, H_ROWS)],
            dwsem,
        )

    def write(r):
        return pltpu.make_async_copy(
            shared.at[lax.rem(r, NBUF)],
            out_hbm.at[pl.ds((c * N_ROUNDS + r) * R_ROWS, S_ROWS)],
            wsem,
        )

    for r0 in range(4):
        idxload(r0).start()
    for r0 in range(3):
        idxload(r0).wait()
        gather(r0).start()
    gather(0).wait()
    copy(0).start()
    dwrite(0).start()

    @pl.loop(0, N_ROUNDS)
    def _round(r):
        @pl.when(jnp.logical_and(r >= NBUF - 1,
                                 s == lax.rem(r - (NBUF - 1), NS)))
        def _():
            write(r - (NBUF - 1)).wait()

        copy(r).wait()
        plsc.subcore_barrier()

        @pl.when(r + 1 < N_ROUNDS)
        def _():
            gather(r + 1).wait()
            copy(r + 1).start()
            dwrite(r + 1).start()

        @pl.when(r + 4 < N_ROUNDS)
        def _():
            idxload(r + 4).start()

        @pl.when(s == lax.rem(r, NS))
        def _():
            write(r).start()

        @pl.when(r + 3 < N_ROUNDS)
        def _():
            dwrite(r).wait()
            idxload(r + 3).wait()
            gather(r + 3).start()

    for r in range(N_ROUNDS - 3, N_ROUNDS):
        dwrite(r).wait()

    for r in range(N_ROUNDS - (NBUF - 1), N_ROUNDS):
        @pl.when(s == lax.rem(jnp.int32(r), NS))
        def _():
            write(r).wait()


def kernel(tokens, table):
    idx5 = tokens.reshape(NC, N_ROUNDS, 2, NS, H_ROWS)
    idx4 = idx5.transpose(0, 1, 3, 2, 4).reshape(NC, N_ROUNDS, NS, T_ROWS)
    mesh = plsc.VectorSubcoreMesh(core_axis_name="c", subcore_axis_name="s")
    out = pl.kernel(
        _body,
        out_type=jax.ShapeDtypeStruct((N_IDX, D), jnp.float32),
        mesh=mesh,
        compiler_params=pltpu.CompilerParams(use_tc_tiling_on_sc=False),
        scratch_types=[
            pltpu.VMEM((4, T_ROWS), jnp.int32),
            pltpu.VMEM((3, T_ROWS, D), jnp.float32),
            pltpu.VMEM_SHARED((NBUF, S_ROWS, D), jnp.float32),
            pltpu.SemaphoreType.DMA((4,)),
            pltpu.SemaphoreType.DMA((3,)),
            pltpu.SemaphoreType.DMA,
            pltpu.SemaphoreType.DMA,
            pltpu.SemaphoreType.DMA,
        ],
    )(idx4, table)
    return out.reshape(BATCH, HIST, D)
